# baseline (speedup 1.0000x reference)
import functools

import jax
import jax.numpy as jnp
from jax import lax
from jax.experimental import pallas as pl
from jax.experimental.pallas import tpu as pltpu
from jax.experimental.pallas import tpu_sc as plsc

_MESH = plsc.VectorSubcoreMesh(core_axis_name="c", subcore_axis_name="s")


@functools.partial(
    pl.kernel,
    out_type=jax.ShapeDtypeStruct((48,), jnp.float32),
    mesh=_MESH,
    scratch_types=[
        pltpu.VMEM((16,), jnp.float32),
        pltpu.VMEM((48,), jnp.float32),
    ],
)
def _scatter_sc(x_hbm, out_hbm, x_v, out_v):
    cid = lax.axis_index("c")
    sid = lax.axis_index("s")

    @pl.when(jnp.logical_and(cid == 0, sid == 0))
    def _():
        zero = jnp.zeros((16,), jnp.float32)
        x_v[...] = zero
        pltpu.sync_copy(x_hbm, x_v.at[pl.ds(0, 12)])
        xv = x_v[...]
        out_v[pl.ds(0, 16)] = zero
        out_v[pl.ds(32, 16)] = zero
        out_v[pl.ds(12, 16)] = xv
        out_v[pl.ds(24, 16)] = xv
        pltpu.sync_copy(out_v, out_hbm)


def kernel(x):
    return _scatter_sc(x.reshape(12)).reshape(4, 2, 2, 3)


# speedup vs baseline: 1.0467x; 1.0467x over previous
import functools

import jax
import jax.numpy as jnp
from jax import lax
from jax.experimental import pallas as pl
from jax.experimental.pallas import tpu as pltpu
from jax.experimental.pallas import tpu_sc as plsc

_MESH = plsc.VectorSubcoreMesh(
    core_axis_name="c", subcore_axis_name="s", num_cores=1
)


@functools.partial(
    pl.kernel,
    out_type=jax.ShapeDtypeStruct((48,), jnp.float32),
    mesh=_MESH,
    scratch_types=[
        pltpu.VMEM((16,), jnp.float32),
        pltpu.VMEM((48,), jnp.float32),
    ],
)
def _scatter_sc(x_hbm, out_hbm, x_v, out_v):
    cid = lax.axis_index("c")
    sid = lax.axis_index("s")

    @pl.when(jnp.logical_and(cid == 0, sid == 0))
    def _():
        zero = jnp.zeros((16,), jnp.float32)
        x_v[...] = zero
        pltpu.sync_copy(x_hbm, x_v.at[pl.ds(0, 12)])
        xv = x_v[...]
        out_v[pl.ds(0, 16)] = zero
        out_v[pl.ds(32, 16)] = zero
        out_v[pl.ds(12, 16)] = xv
        out_v[pl.ds(24, 16)] = xv
        pltpu.sync_copy(out_v, out_hbm)


def kernel(x):
    return _scatter_sc(x.reshape(12)).reshape(4, 2, 2, 3)


# speedup vs baseline: 1.1244x; 1.0742x over previous
import functools

import jax
import jax.numpy as jnp
from jax import lax
from jax.experimental import pallas as pl
from jax.experimental.pallas import tpu as pltpu
from jax.experimental.pallas import tpu_sc as plsc

_MESH = plsc.ScalarSubcoreMesh(axis_name="c", num_cores=1)


@functools.partial(
    pl.kernel,
    out_type=jax.ShapeDtypeStruct((4, 12), jnp.float32),
    mesh=_MESH,
    scratch_types=[pltpu.SemaphoreType.DMA],
)
def _scatter_sc(x_hbm, z_hbm, out_hbm, sem):
    c0 = pltpu.async_copy(z_hbm, out_hbm.at[0], sem)
    c1 = pltpu.async_copy(x_hbm, out_hbm.at[1], sem)
    c2 = pltpu.async_copy(x_hbm, out_hbm.at[2], sem)
    c3 = pltpu.async_copy(z_hbm, out_hbm.at[3], sem)
    c0.wait()
    c1.wait()
    c2.wait()
    c3.wait()


def kernel(x):
    zeros = jnp.zeros((12,), jnp.float32)
    return _scatter_sc(x.reshape(12), zeros).reshape(4, 2, 2, 3)
